# baseline (speedup 1.0000x reference)
import jax
import jax.numpy as jnp
from jax.experimental import pallas as pl
from jax.experimental.pallas import tpu as pltpu

BATCH = 8
N_TILES = 4
N_TOKENS = 1025
EMBED_DIM = 1280
MAX_NUM_TILES = 4


def _i32(v):
    return jnp.asarray(v, dtype=jnp.int32)


def _body(ar_ref, gate_ref, x_ref, emb_ref, out_ref):
    bt = pl.program_id(0)
    b = bt // N_TILES
    t = bt % N_TILES
    h = ar_ref[2 * b]
    w = ar_ref[2 * b + 1]
    n = h * w
    valid = t < n
    w_safe = jnp.maximum(w, 1)
    i = jnp.where(valid, t // w_safe, 0)
    j = jnp.where(valid, t % w_safe, 0)
    row = emb_ref[pl.ds(i * MAX_NUM_TILES + j, 1), :]
    gate_t = jnp.tanh(gate_ref[0])
    coef = jnp.where(valid, gate_t, 0.0)
    out_ref[:] = x_ref[:] + (coef * row)[None, :, :]


def kernel(x, aspect_ratio, embedding, gate):
    ar = aspect_ratio.astype(jnp.int32).reshape(-1)
    xf = x.reshape(BATCH * N_TILES, N_TOKENS, EMBED_DIM)
    emb = embedding.reshape(MAX_NUM_TILES * MAX_NUM_TILES, EMBED_DIM)

    grid_spec = pltpu.PrefetchScalarGridSpec(
        num_scalar_prefetch=2,
        grid=(BATCH * N_TILES, 1, 1),
        in_specs=[
            pl.BlockSpec((1, N_TOKENS, EMBED_DIM),
                         lambda bt, z0, z1, ar, g: (bt, z0, z1)),
            pl.BlockSpec((MAX_NUM_TILES * MAX_NUM_TILES, EMBED_DIM),
                         lambda bt, z0, z1, ar, g: (z0, z1)),
        ],
        out_specs=pl.BlockSpec((1, N_TOKENS, EMBED_DIM),
                               lambda bt, z0, z1, ar, g: (bt, z0, z1)),
    )

    out = pl.pallas_call(
        _body,
        grid_spec=grid_spec,
        out_shape=jax.ShapeDtypeStruct(xf.shape, xf.dtype),
    )(ar, gate.astype(jnp.float32), xf, emb)
    return out.reshape(x.shape)


# speedup vs baseline: 3.3258x; 3.3258x over previous
import jax
import jax.numpy as jnp
from jax.experimental import pallas as pl
from jax.experimental.pallas import tpu as pltpu

BATCH = 8
N_TILES = 4
N_TOKENS = 1025
EMBED_DIM = 1280
MAX_NUM_TILES = 4


def _body(ar_ref, gate_ref, x_ref, emb_ref, out_ref):
    b = pl.program_id(0)
    t = pl.program_id(1)
    h = ar_ref[2 * b]
    w = ar_ref[2 * b + 1]
    n = h * w
    valid = t < n
    w_safe = jnp.maximum(w, 1)
    i = jnp.where(valid, t // w_safe, 0)
    j = jnp.where(valid, t % w_safe, 0)
    row = emb_ref[i, j]
    gate_t = jnp.tanh(gate_ref[0])
    coef = jnp.where(valid, gate_t, 0.0)
    out_ref[:] = x_ref[:] + (coef * row)[None, None, :, :]


def kernel(x, aspect_ratio, embedding, gate):
    ar = aspect_ratio.astype(jnp.int32).reshape(-1)

    grid_spec = pltpu.PrefetchScalarGridSpec(
        num_scalar_prefetch=2,
        grid=(BATCH, N_TILES, 1, 1),
        in_specs=[
            pl.BlockSpec((1, 1, N_TOKENS, EMBED_DIM),
                         lambda b, t, z0, z1, ar, g: (b, t, z0, z1)),
            pl.BlockSpec((MAX_NUM_TILES, MAX_NUM_TILES, 1, EMBED_DIM),
                         lambda b, t, z0, z1, ar, g: (z0, z1, z0, z1)),
        ],
        out_specs=pl.BlockSpec((1, 1, N_TOKENS, EMBED_DIM),
                               lambda b, t, z0, z1, ar, g: (b, t, z0, z1)),
    )

    out = pl.pallas_call(
        _body,
        grid_spec=grid_spec,
        out_shape=jax.ShapeDtypeStruct(x.shape, x.dtype),
    )(ar, gate.astype(jnp.float32), x, embedding)
    return out
